# baseline (speedup 1.0000x reference)
import functools

import jax
import jax.numpy as jnp
from jax import lax
from jax.experimental import pallas as pl
from jax.experimental.pallas import tpu as pltpu
from jax.experimental.pallas import tpu_sc as plsc

N = 10000
E = 320000
C = 128
NP = 10240
CHUNK = 80
DEG_ROWS_PER_TILE = (E // CHUNK) // 16
W_ROWS_PER_WORKER = (E // 16) // 32
SPMM_ROWS_PER_WORKER = (E // CHUNK) // 32
NPT = NP // 16

_MESH = plsc.VectorSubcoreMesh(
    core_axis_name="c", subcore_axis_name="s", num_cores=2, num_subcores=16
)


@functools.partial(
    pl.kernel,
    out_type=jax.ShapeDtypeStruct((32, SPMM_ROWS_PER_WORKER, CHUNK), jnp.float32),
    mesh=_MESH,
    compiler_params=pltpu.CompilerParams(needs_layout_passes=False, use_tc_tiling_on_sc=False),
    scratch_types=[
        pltpu.VMEM((SPMM_ROWS_PER_WORKER, CHUNK), jnp.int32),
        pltpu.VMEM((SPMM_ROWS_PER_WORKER, CHUNK), jnp.int32),
        pltpu.VMEM((SPMM_ROWS_PER_WORKER, CHUNK), jnp.float32),
        pltpu.VMEM((CHUNK,), jnp.float32),
        pltpu.VMEM((NPT,), jnp.float32),
        pltpu.VMEM((NP,), jnp.float32),
        pltpu.VMEM_SHARED((NP,), jnp.float32),
        pltpu.VMEM_SHARED((NP,), jnp.float32),
    ],
)
def _prepass(row3_hbm, col3_hbm, a_hbm,
             rowv80, colv80, av, onesv, degv, disv, deg_sh, dis_sh):
    c = lax.axis_index("c")
    s = lax.axis_index("s")
    wid = c * 16 + s

    for i in range(CHUNK // 16):
        onesv[pl.ds(i * 16, 16)] = jnp.full((16,), 1.0, jnp.float32)
    for i in range(NPT // 16):
        degv[pl.ds(i * 16, 16)] = jnp.zeros((16,), jnp.float32)
    pltpu.sync_copy(degv, deg_sh.at[pl.ds(s * NPT, NPT)])
    plsc.subcore_barrier()

    def deg_body(j, carry):
        pltpu.sync_copy(onesv, deg_sh.at[rowv80.at[j]], add=True)
        return carry

    for half in range(2):
        pltpu.sync_copy(row3_hbm.at[s * 2 + half], rowv80)
        lax.fori_loop(0, SPMM_ROWS_PER_WORKER, deg_body, 0)
    plsc.subcore_barrier()

    pltpu.sync_copy(deg_sh.at[pl.ds(s * NPT, NPT)], degv)
    for i in range(NPT // 16):
        d = degv[pl.ds(i * 16, 16)]
        dsafe = jnp.maximum(d, 1.0)
        sq = (dsafe + 1.0) * 0.5
        for _ in range(14):
            sq = (sq + dsafe / sq) * 0.5
        degv[pl.ds(i * 16, 16)] = jnp.where(
            d > 0.0, 1.0 / sq, jnp.zeros((16,), jnp.float32))
    pltpu.sync_copy(degv, dis_sh.at[pl.ds(s * NPT, NPT)])
    plsc.subcore_barrier()

    pltpu.sync_copy(dis_sh, disv)
    pltpu.sync_copy(row3_hbm.at[wid], rowv80)
    pltpu.sync_copy(col3_hbm.at[wid], colv80)

    def w_body(j, carry):
        for k in range(CHUNK // 16):
            dr = plsc.load_gather(disv, [rowv80[j, pl.ds(k * 16, 16)]])
            dc = plsc.load_gather(disv, [colv80[j, pl.ds(k * 16, 16)]])
            av[j, pl.ds(k * 16, 16)] = -(dr * dc) - 1.0
        return carry

    lax.fori_loop(0, SPMM_ROWS_PER_WORKER, w_body, 0)
    pltpu.sync_copy(av, a_hbm.at[wid])


@functools.partial(
    pl.kernel,
    out_type=jax.ShapeDtypeStruct((2, 16, 625, C), jnp.float32),
    mesh=_MESH,
    compiler_params=pltpu.CompilerParams(needs_layout_passes=False, use_tc_tiling_on_sc=False),
    scratch_types=[
        pltpu.VMEM((SPMM_ROWS_PER_WORKER, CHUNK), jnp.int32),
        pltpu.VMEM((SPMM_ROWS_PER_WORKER, CHUNK), jnp.int32),
        pltpu.VMEM((SPMM_ROWS_PER_WORKER, CHUNK), jnp.float32),
        pltpu.VMEM((CHUNK, C), jnp.float32),
        pltpu.VMEM((CHUNK, C), jnp.float32),
        pltpu.VMEM_SHARED((N, C), jnp.float32),
        pltpu.SemaphoreType.DMA,
        pltpu.SemaphoreType.DMA,
        pltpu.SemaphoreType.DMA,
        pltpu.SemaphoreType.DMA,
    ],
)
def _spmm(h_hbm, col3_hbm, row3_hbm, a3_hbm, part_hbm,
          colv, rowv, av, gbuf0, gbuf1, acc_sh, sem0, sem1, ssem0, ssem1):
    c = lax.axis_index("c")
    s = lax.axis_index("s")
    wid = c * 16 + s

    for i in range(CHUNK):
        for r in range(C // 16):
            gbuf0[i, pl.ds(r * 16, 16)] = jnp.zeros((16,), jnp.float32)
    for i in range(7):
        pltpu.sync_copy(gbuf0, acc_sh.at[pl.ds(s * 625 + i * 80, 80)])
    pltpu.sync_copy(gbuf0.at[pl.ds(0, 65)], acc_sh.at[pl.ds(s * 625 + 560, 65)])
    plsc.subcore_barrier()

    pltpu.sync_copy(col3_hbm.at[wid], colv)
    pltpu.sync_copy(row3_hbm.at[wid], rowv)
    pltpu.sync_copy(a3_hbm.at[wid], av)

    def scale(m, gbuf):
        for k in range(CHUNK // 16):
            avv = av[m, pl.ds(k * 16, 16)]
            for t in range(16):
                jj = k * 16 + t
                aa = avv[t]
                for r in range(C // 16):
                    gbuf[jj, pl.ds(r * 16, 16)] = gbuf[jj, pl.ds(r * 16, 16)] * aa

    def wait_gather(m, gbuf, sem):
        pltpu.make_async_copy(h_hbm.at[colv.at[m]], gbuf, sem).wait()

    def wait_scatter(m, gbuf, ssem):
        pltpu.make_async_copy(gbuf, acc_sh.at[rowv.at[m]], ssem).wait()

    pltpu.async_copy(h_hbm.at[colv.at[0]], gbuf0, sem0)
    wait_gather(0, gbuf0, sem0)
    scale(0, gbuf0)
    pltpu.async_copy(h_hbm.at[colv.at[1]], gbuf1, sem1)
    pltpu.async_copy(gbuf0, acc_sh.at[rowv.at[0]], ssem0, add=True)

    def body(j, carry):
        mA = 2 * j + 1
        wait_gather(mA, gbuf1, sem1)
        scale(mA, gbuf1)
        wait_scatter(mA - 1, gbuf0, ssem0)
        pltpu.async_copy(h_hbm.at[colv.at[mA + 1]], gbuf0, sem0)
        pltpu.async_copy(gbuf1, acc_sh.at[rowv.at[mA]], ssem1, add=True)
        wait_gather(mA + 1, gbuf0, sem0)
        scale(mA + 1, gbuf0)
        wait_scatter(mA, gbuf1, ssem1)
        pltpu.async_copy(h_hbm.at[colv.at[mA + 2]], gbuf1, sem1)
        pltpu.async_copy(gbuf0, acc_sh.at[rowv.at[mA + 1]], ssem0, add=True)
        return carry

    lax.fori_loop(0, 61, body, 0)
    wait_gather(123, gbuf1, sem1)
    scale(123, gbuf1)
    wait_scatter(122, gbuf0, ssem0)
    pltpu.async_copy(h_hbm.at[colv.at[124]], gbuf0, sem0)
    pltpu.async_copy(gbuf1, acc_sh.at[rowv.at[123]], ssem1, add=True)
    wait_gather(124, gbuf0, sem0)
    scale(124, gbuf0)
    wait_scatter(123, gbuf1, ssem1)
    pltpu.async_copy(gbuf0, acc_sh.at[rowv.at[124]], ssem0, add=True)
    wait_scatter(124, gbuf0, ssem0)
    plsc.subcore_barrier()

    pltpu.sync_copy(acc_sh.at[pl.ds(s * 625, 625)], part_hbm.at[c, s])


def _combine_body(p_ref, o_ref):
    o_ref[...] = p_ref[0] + p_ref[1]


def _final_body(x_ref, t1_ref, q_ref, w_ref, b_ref, o_ref):
    s2 = q_ref[0] + q_ref[1]
    acc = jnp.dot(x_ref[...], w_ref[0], preferred_element_type=jnp.float32)
    acc = acc + jnp.dot(t1_ref[...], w_ref[1], preferred_element_type=jnp.float32)
    acc = acc + jnp.dot(s2, w_ref[2], preferred_element_type=jnp.float32)
    o_ref[...] = acc + b_ref[...]


def kernel(x, edge_index, W, b):
    row = edge_index[0]
    col = edge_index[1]
    row3 = row.reshape(32, SPMM_ROWS_PER_WORKER, CHUNK)
    col3 = col.reshape(32, SPMM_ROWS_PER_WORKER, CHUNK)

    a3 = _prepass(row3, col3)

    p = _spmm(x, col3, row3, a3).reshape(2, N, C)
    t1 = pl.pallas_call(
        _combine_body,
        grid=(10,),
        in_specs=[pl.BlockSpec((2, N // 10, C), lambda i: (0, i, 0))],
        out_specs=pl.BlockSpec((N // 10, C), lambda i: (i, 0)),
        out_shape=jax.ShapeDtypeStruct((N, C), jnp.float32),
    )(p)

    q = _spmm(t1, col3, row3, a3).reshape(2, N, C)

    Wc = jnp.stack([W[0] - W[2], W[1], 2.0 * W[2]])
    b2 = b.reshape(1, C)
    out = pl.pallas_call(
        _final_body,
        grid=(10,),
        in_specs=[
            pl.BlockSpec((N // 10, C), lambda i: (i, 0)),
            pl.BlockSpec((N // 10, C), lambda i: (i, 0)),
            pl.BlockSpec((2, N // 10, C), lambda i: (0, i, 0)),
            pl.BlockSpec((3, C, C), lambda i: (0, 0, 0)),
            pl.BlockSpec((1, C), lambda i: (0, 0)),
        ],
        out_specs=pl.BlockSpec((N // 10, C), lambda i: (i, 0)),
        out_shape=jax.ShapeDtypeStruct((N, C), jnp.float32),
    )(x, t1, q, Wc, b2)
    return out


# speedup vs baseline: 1.3661x; 1.3661x over previous
import functools

import jax
import jax.numpy as jnp
from jax import lax
from jax.experimental import pallas as pl
from jax.experimental.pallas import tpu as pltpu
from jax.experimental.pallas import tpu_sc as plsc

N = 10000
E = 320000
C = 128
NP = 10240
CHUNK = 80
DEG_ROWS_PER_TILE = (E // CHUNK) // 16
W_ROWS_PER_WORKER = (E // 16) // 32
SPMM_ROWS_PER_WORKER = (E // CHUNK) // 32
NPT = NP // 16

_MESH = plsc.VectorSubcoreMesh(
    core_axis_name="c", subcore_axis_name="s", num_cores=2, num_subcores=16
)


@functools.partial(
    pl.kernel,
    out_type=jax.ShapeDtypeStruct((32, SPMM_ROWS_PER_WORKER, CHUNK), jnp.float32),
    mesh=_MESH,
    compiler_params=pltpu.CompilerParams(needs_layout_passes=False, use_tc_tiling_on_sc=False),
    scratch_types=[
        pltpu.VMEM((SPMM_ROWS_PER_WORKER, CHUNK), jnp.int32),
        pltpu.VMEM((SPMM_ROWS_PER_WORKER, CHUNK), jnp.int32),
        pltpu.VMEM((SPMM_ROWS_PER_WORKER, CHUNK), jnp.float32),
        pltpu.VMEM((CHUNK,), jnp.float32),
        pltpu.VMEM((NPT,), jnp.float32),
        pltpu.VMEM((NP,), jnp.float32),
        pltpu.VMEM_SHARED((NP,), jnp.float32),
        pltpu.VMEM_SHARED((NP,), jnp.float32),
    ],
)
def _prepass(row3_hbm, col3_hbm, a_hbm,
             rowv80, colv80, av, onesv, degv, disv, deg_sh, dis_sh):
    c = lax.axis_index("c")
    s = lax.axis_index("s")
    wid = c * 16 + s

    for i in range(CHUNK // 16):
        onesv[pl.ds(i * 16, 16)] = jnp.full((16,), 1.0, jnp.float32)
    for i in range(NPT // 16):
        degv[pl.ds(i * 16, 16)] = jnp.zeros((16,), jnp.float32)
    pltpu.sync_copy(degv, deg_sh.at[pl.ds(s * NPT, NPT)])
    plsc.subcore_barrier()

    def deg_body(j, carry):
        pltpu.sync_copy(onesv, deg_sh.at[rowv80.at[j]], add=True)
        return carry

    for half in range(2):
        pltpu.sync_copy(row3_hbm.at[s * 2 + half], rowv80)
        lax.fori_loop(0, SPMM_ROWS_PER_WORKER, deg_body, 0)
    plsc.subcore_barrier()

    pltpu.sync_copy(deg_sh.at[pl.ds(s * NPT, NPT)], degv)
    for i in range(NPT // 16):
        d = degv[pl.ds(i * 16, 16)]
        dsafe = jnp.maximum(d, 1.0)
        sq = (dsafe + 1.0) * 0.5
        for _ in range(14):
            sq = (sq + dsafe / sq) * 0.5
        degv[pl.ds(i * 16, 16)] = jnp.where(
            d > 0.0, 1.0 / sq, jnp.zeros((16,), jnp.float32))
    pltpu.sync_copy(degv, dis_sh.at[pl.ds(s * NPT, NPT)])
    plsc.subcore_barrier()

    pltpu.sync_copy(dis_sh, disv)
    pltpu.sync_copy(row3_hbm.at[wid], rowv80)
    pltpu.sync_copy(col3_hbm.at[wid], colv80)

    def w_body(j, carry):
        for k in range(CHUNK // 16):
            dr = plsc.load_gather(disv, [rowv80[j, pl.ds(k * 16, 16)]])
            dc = plsc.load_gather(disv, [colv80[j, pl.ds(k * 16, 16)]])
            av[j, pl.ds(k * 16, 16)] = -(dr * dc) - 1.0
        return carry

    lax.fori_loop(0, SPMM_ROWS_PER_WORKER, w_body, 0)
    pltpu.sync_copy(av, a_hbm.at[wid])


@functools.partial(
    pl.kernel,
    out_type=jax.ShapeDtypeStruct((2, 16, 625, C), jnp.float32),
    mesh=_MESH,
    compiler_params=pltpu.CompilerParams(needs_layout_passes=False, use_tc_tiling_on_sc=False),
    scratch_types=[
        pltpu.VMEM((SPMM_ROWS_PER_WORKER, CHUNK), jnp.int32),
        pltpu.VMEM((SPMM_ROWS_PER_WORKER, 2, CHUNK // 2), jnp.int32),
        pltpu.VMEM((SPMM_ROWS_PER_WORKER, CHUNK), jnp.float32),
        pltpu.VMEM((CHUNK, C), jnp.float32),
        pltpu.VMEM((CHUNK, C), jnp.float32),
        pltpu.VMEM_SHARED((N, C), jnp.float32),
        pltpu.SemaphoreType.DMA,
        pltpu.SemaphoreType.DMA,
        pltpu.SemaphoreType.DMA,
        pltpu.SemaphoreType.DMA,
    ],
)
def _spmm(h_hbm, col3_hbm, row3_hbm, a3_hbm, part_hbm,
          colv, rowv, av, gbuf0, gbuf1, acc_sh, sem0, sem1, ssem0, ssem1):
    c = lax.axis_index("c")
    s = lax.axis_index("s")
    wid = c * 16 + s

    for i in range(CHUNK):
        for r in range(C // 16):
            gbuf0[i, pl.ds(r * 16, 16)] = jnp.zeros((16,), jnp.float32)
    for i in range(7):
        pltpu.sync_copy(gbuf0, acc_sh.at[pl.ds(s * 625 + i * 80, 80)])
    pltpu.sync_copy(gbuf0.at[pl.ds(0, 65)], acc_sh.at[pl.ds(s * 625 + 560, 65)])
    plsc.subcore_barrier()

    pltpu.sync_copy(col3_hbm.at[wid], colv)
    pltpu.sync_copy(row3_hbm.at[wid], rowv)
    pltpu.sync_copy(a3_hbm.at[wid], av)

    HALF = CHUNK // 2

    def scale_half(m, h, gbuf):
        for off, lo in ((0, 0), (16, 0), (24, 8)):
            avv = av[m, pl.ds(h * HALF + off, 16)]
            for t in range(lo, 16):
                jj = h * HALF + off + t
                aa = avv[t]
                for r in range(C // 16):
                    gbuf[jj, pl.ds(r * 16, 16)] = gbuf[jj, pl.ds(r * 16, 16)] * aa

    def wait_gather(m, gbuf, sem):
        pltpu.make_async_copy(h_hbm.at[colv.at[m]], gbuf, sem).wait()

    def process(m, gbuf, sem, ssem):
        wait_gather(m, gbuf, sem)
        scale_half(m, 0, gbuf)
        pltpu.async_copy(gbuf.at[pl.ds(0, HALF)], acc_sh.at[rowv.at[m, 0]],
                         ssem, add=True)
        scale_half(m, 1, gbuf)
        pltpu.async_copy(gbuf.at[pl.ds(HALF, HALF)], acc_sh.at[rowv.at[m, 1]],
                         ssem, add=True)
        pltpu.make_async_copy(gbuf.at[pl.ds(0, HALF)], acc_sh.at[rowv.at[m, 0]],
                              ssem).wait()
        pltpu.make_async_copy(gbuf.at[pl.ds(HALF, HALF)], acc_sh.at[rowv.at[m, 1]],
                              ssem).wait()

    pltpu.async_copy(h_hbm.at[colv.at[0]], gbuf0, sem0)

    def body(j, carry):
        m0 = 2 * j
        pltpu.async_copy(h_hbm.at[colv.at[m0 + 1]], gbuf1, sem1)
        process(m0, gbuf0, sem0, ssem0)
        pltpu.async_copy(h_hbm.at[colv.at[m0 + 2]], gbuf0, sem0)
        process(m0 + 1, gbuf1, sem1, ssem1)
        return carry

    lax.fori_loop(0, (SPMM_ROWS_PER_WORKER - 1) // 2, body, 0)
    process(SPMM_ROWS_PER_WORKER - 1, gbuf0, sem0, ssem0)
    plsc.subcore_barrier()

    pltpu.sync_copy(acc_sh.at[pl.ds(s * 625, 625)], part_hbm.at[c, s])


def _combine_body(p_ref, o_ref):
    o_ref[...] = p_ref[0] + p_ref[1]


def _final_body(x_ref, t1_ref, q_ref, w_ref, b_ref, o_ref):
    s2 = q_ref[0] + q_ref[1]
    acc = jnp.dot(x_ref[...], w_ref[0], preferred_element_type=jnp.float32)
    acc = acc + jnp.dot(t1_ref[...], w_ref[1], preferred_element_type=jnp.float32)
    acc = acc + jnp.dot(s2, w_ref[2], preferred_element_type=jnp.float32)
    o_ref[...] = acc + b_ref[...]


def kernel(x, edge_index, W, b):
    row = edge_index[0]
    col = edge_index[1]
    row3 = row.reshape(32, SPMM_ROWS_PER_WORKER, CHUNK)
    col3 = col.reshape(32, SPMM_ROWS_PER_WORKER, CHUNK)

    a3 = _prepass(row3, col3)

    row4 = row.reshape(32, SPMM_ROWS_PER_WORKER, 2, CHUNK // 2)
    p = _spmm(x, col3, row4, a3).reshape(2, N, C)
    t1 = pl.pallas_call(
        _combine_body,
        grid=(10,),
        in_specs=[pl.BlockSpec((2, N // 10, C), lambda i: (0, i, 0))],
        out_specs=pl.BlockSpec((N // 10, C), lambda i: (i, 0)),
        out_shape=jax.ShapeDtypeStruct((N, C), jnp.float32),
    )(p)

    q = _spmm(t1, col3, row4, a3).reshape(2, N, C)

    Wc = jnp.stack([W[0] - W[2], W[1], 2.0 * W[2]])
    b2 = b.reshape(1, C)
    out = pl.pallas_call(
        _final_body,
        grid=(10,),
        in_specs=[
            pl.BlockSpec((N // 10, C), lambda i: (i, 0)),
            pl.BlockSpec((N // 10, C), lambda i: (i, 0)),
            pl.BlockSpec((2, N // 10, C), lambda i: (0, i, 0)),
            pl.BlockSpec((3, C, C), lambda i: (0, 0, 0)),
            pl.BlockSpec((1, C), lambda i: (0, 0)),
        ],
        out_specs=pl.BlockSpec((N // 10, C), lambda i: (i, 0)),
        out_shape=jax.ShapeDtypeStruct((N, C), jnp.float32),
    )(x, t1, q, Wc, b2)
    return out


# speedup vs baseline: 1.3977x; 1.0231x over previous
import functools

import jax
import jax.numpy as jnp
from jax import lax
from jax.experimental import pallas as pl
from jax.experimental.pallas import tpu as pltpu
from jax.experimental.pallas import tpu_sc as plsc

N = 10000
E = 320000
C = 128
NP = 10240
CHUNK = 80
DEG_ROWS_PER_TILE = (E // CHUNK) // 16
W_ROWS_PER_WORKER = (E // 16) // 32
SPMM_ROWS_PER_WORKER = (E // CHUNK) // 32
NPT = NP // 16

_MESH = plsc.VectorSubcoreMesh(
    core_axis_name="c", subcore_axis_name="s", num_cores=2, num_subcores=16
)


@functools.partial(
    pl.kernel,
    out_type=jax.ShapeDtypeStruct((32, SPMM_ROWS_PER_WORKER, CHUNK), jnp.float32),
    mesh=_MESH,
    compiler_params=pltpu.CompilerParams(needs_layout_passes=False, use_tc_tiling_on_sc=False),
    scratch_types=[
        pltpu.VMEM((SPMM_ROWS_PER_WORKER, CHUNK), jnp.int32),
        pltpu.VMEM((SPMM_ROWS_PER_WORKER, CHUNK), jnp.int32),
        pltpu.VMEM((SPMM_ROWS_PER_WORKER, CHUNK), jnp.float32),
        pltpu.VMEM((CHUNK,), jnp.float32),
        pltpu.VMEM((NPT,), jnp.float32),
        pltpu.VMEM((NP,), jnp.float32),
        pltpu.VMEM_SHARED((NP,), jnp.float32),
        pltpu.VMEM_SHARED((NP,), jnp.float32),
    ],
)
def _prepass(row3_hbm, col3_hbm, a_hbm,
             rowv80, colv80, av, onesv, degv, disv, deg_sh, dis_sh):
    c = lax.axis_index("c")
    s = lax.axis_index("s")
    wid = c * 16 + s

    for i in range(CHUNK // 16):
        onesv[pl.ds(i * 16, 16)] = jnp.full((16,), 1.0, jnp.float32)
    for i in range(NPT // 16):
        degv[pl.ds(i * 16, 16)] = jnp.zeros((16,), jnp.float32)
    pltpu.sync_copy(degv, deg_sh.at[pl.ds(s * NPT, NPT)])
    plsc.subcore_barrier()

    def deg_body(j, carry):
        pltpu.sync_copy(onesv, deg_sh.at[rowv80.at[j]], add=True)
        return carry

    for half in range(2):
        pltpu.sync_copy(row3_hbm.at[s * 2 + half], rowv80)
        lax.fori_loop(0, SPMM_ROWS_PER_WORKER, deg_body, 0)
    plsc.subcore_barrier()

    pltpu.sync_copy(deg_sh.at[pl.ds(s * NPT, NPT)], degv)
    for i in range(NPT // 16):
        d = degv[pl.ds(i * 16, 16)]
        dsafe = jnp.maximum(d, 1.0)
        sq = (dsafe + 1.0) * 0.5
        for _ in range(14):
            sq = (sq + dsafe / sq) * 0.5
        degv[pl.ds(i * 16, 16)] = jnp.where(
            d > 0.0, 1.0 / sq, jnp.zeros((16,), jnp.float32))
    pltpu.sync_copy(degv, dis_sh.at[pl.ds(s * NPT, NPT)])
    plsc.subcore_barrier()

    pltpu.sync_copy(dis_sh, disv)
    pltpu.sync_copy(row3_hbm.at[wid], rowv80)
    pltpu.sync_copy(col3_hbm.at[wid], colv80)

    def w_body(j, carry):
        for k in range(CHUNK // 16):
            dr = plsc.load_gather(disv, [rowv80[j, pl.ds(k * 16, 16)]])
            dc = plsc.load_gather(disv, [colv80[j, pl.ds(k * 16, 16)]])
            av[j, pl.ds(k * 16, 16)] = -(dr * dc) - 1.0
        return carry

    lax.fori_loop(0, SPMM_ROWS_PER_WORKER, w_body, 0)
    pltpu.sync_copy(av, a_hbm.at[wid])


@functools.partial(
    pl.kernel,
    out_type=jax.ShapeDtypeStruct((2, 16, 625, C), jnp.float32),
    mesh=_MESH,
    compiler_params=pltpu.CompilerParams(needs_layout_passes=False, use_tc_tiling_on_sc=False),
    scratch_types=[
        pltpu.VMEM((SPMM_ROWS_PER_WORKER, 2, CHUNK // 2), jnp.int32),
        pltpu.VMEM((SPMM_ROWS_PER_WORKER, 2, CHUNK // 2), jnp.int32),
        pltpu.VMEM((SPMM_ROWS_PER_WORKER, CHUNK), jnp.float32),
        pltpu.VMEM((CHUNK, C), jnp.float32),
        pltpu.VMEM((CHUNK, C), jnp.float32),
        pltpu.VMEM_SHARED((N, C), jnp.float32),
        pltpu.SemaphoreType.DMA,
        pltpu.SemaphoreType.DMA,
        pltpu.SemaphoreType.DMA,
        pltpu.SemaphoreType.DMA,
        pltpu.SemaphoreType.DMA,
        pltpu.SemaphoreType.DMA,
    ],
)
def _spmm(h_hbm, col3_hbm, row3_hbm, a3_hbm, part_hbm,
          colv, rowv, av, gbuf0, gbuf1, acc_sh,
          sem0a, sem0b, sem1a, sem1b, ssem0, ssem1):
    c = lax.axis_index("c")
    s = lax.axis_index("s")
    wid = c * 16 + s

    for i in range(CHUNK):
        for r in range(C // 16):
            gbuf0[i, pl.ds(r * 16, 16)] = jnp.zeros((16,), jnp.float32)
    for i in range(7):
        pltpu.sync_copy(gbuf0, acc_sh.at[pl.ds(s * 625 + i * 80, 80)])
    pltpu.sync_copy(gbuf0.at[pl.ds(0, 65)], acc_sh.at[pl.ds(s * 625 + 560, 65)])
    plsc.subcore_barrier()

    pltpu.sync_copy(col3_hbm.at[wid], colv)
    pltpu.sync_copy(row3_hbm.at[wid], rowv)
    pltpu.sync_copy(a3_hbm.at[wid], av)

    HALF = CHUNK // 2

    def scale_half(m, h, gbuf):
        for off, lo in ((0, 0), (16, 0), (24, 8)):
            avv = av[m, pl.ds(h * HALF + off, 16)]
            for t in range(lo, 16):
                jj = h * HALF + off + t
                aa = avv[t]
                for r in range(C // 16):
                    gbuf[jj, pl.ds(r * 16, 16)] = gbuf[jj, pl.ds(r * 16, 16)] * aa

    def issue_gather(m, gbuf, sema, semb):
        pltpu.async_copy(h_hbm.at[colv.at[m, 0]], gbuf.at[pl.ds(0, HALF)], sema)
        pltpu.async_copy(h_hbm.at[colv.at[m, 1]], gbuf.at[pl.ds(HALF, HALF)], semb)

    def process(m, gbuf, sema, semb, ssem):
        pltpu.make_async_copy(h_hbm.at[colv.at[m, 0]], gbuf.at[pl.ds(0, HALF)],
                              sema).wait()
        scale_half(m, 0, gbuf)
        pltpu.async_copy(gbuf.at[pl.ds(0, HALF)], acc_sh.at[rowv.at[m, 0]],
                         ssem, add=True)
        pltpu.make_async_copy(h_hbm.at[colv.at[m, 1]], gbuf.at[pl.ds(HALF, HALF)],
                              semb).wait()
        scale_half(m, 1, gbuf)
        pltpu.async_copy(gbuf.at[pl.ds(HALF, HALF)], acc_sh.at[rowv.at[m, 1]],
                         ssem, add=True)
        pltpu.make_async_copy(gbuf.at[pl.ds(0, HALF)], acc_sh.at[rowv.at[m, 0]],
                              ssem).wait()
        pltpu.make_async_copy(gbuf.at[pl.ds(HALF, HALF)], acc_sh.at[rowv.at[m, 1]],
                              ssem).wait()

    issue_gather(0, gbuf0, sem0a, sem0b)

    def body(j, carry):
        m0 = 2 * j
        issue_gather(m0 + 1, gbuf1, sem1a, sem1b)
        process(m0, gbuf0, sem0a, sem0b, ssem0)
        issue_gather(m0 + 2, gbuf0, sem0a, sem0b)
        process(m0 + 1, gbuf1, sem1a, sem1b, ssem1)
        return carry

    lax.fori_loop(0, (SPMM_ROWS_PER_WORKER - 1) // 2, body, 0)
    process(SPMM_ROWS_PER_WORKER - 1, gbuf0, sem0a, sem0b, ssem0)
    plsc.subcore_barrier()

    pltpu.sync_copy(acc_sh.at[pl.ds(s * 625, 625)], part_hbm.at[c, s])


def _combine_body(p_ref, o_ref):
    o_ref[...] = p_ref[0] + p_ref[1]


def _final_body(x_ref, t1_ref, q_ref, w_ref, b_ref, o_ref):
    s2 = q_ref[0] + q_ref[1]
    acc = jnp.dot(x_ref[...], w_ref[0], preferred_element_type=jnp.float32)
    acc = acc + jnp.dot(t1_ref[...], w_ref[1], preferred_element_type=jnp.float32)
    acc = acc + jnp.dot(s2, w_ref[2], preferred_element_type=jnp.float32)
    o_ref[...] = acc + b_ref[...]


def kernel(x, edge_index, W, b):
    row = edge_index[0]
    col = edge_index[1]
    row3 = row.reshape(32, SPMM_ROWS_PER_WORKER, CHUNK)
    col3 = col.reshape(32, SPMM_ROWS_PER_WORKER, CHUNK)

    a3 = _prepass(row3, col3)

    row4 = row.reshape(32, SPMM_ROWS_PER_WORKER, 2, CHUNK // 2)
    col4 = col.reshape(32, SPMM_ROWS_PER_WORKER, 2, CHUNK // 2)
    p = _spmm(x, col4, row4, a3).reshape(2, N, C)
    t1 = pl.pallas_call(
        _combine_body,
        grid=(10,),
        in_specs=[pl.BlockSpec((2, N // 10, C), lambda i: (0, i, 0))],
        out_specs=pl.BlockSpec((N // 10, C), lambda i: (i, 0)),
        out_shape=jax.ShapeDtypeStruct((N, C), jnp.float32),
    )(p)

    q = _spmm(t1, col4, row4, a3).reshape(2, N, C)

    Wc = jnp.stack([W[0] - W[2], W[1], 2.0 * W[2]])
    b2 = b.reshape(1, C)
    out = pl.pallas_call(
        _final_body,
        grid=(10,),
        in_specs=[
            pl.BlockSpec((N // 10, C), lambda i: (i, 0)),
            pl.BlockSpec((N // 10, C), lambda i: (i, 0)),
            pl.BlockSpec((2, N // 10, C), lambda i: (0, i, 0)),
            pl.BlockSpec((3, C, C), lambda i: (0, 0, 0)),
            pl.BlockSpec((1, C), lambda i: (0, 0)),
        ],
        out_specs=pl.BlockSpec((N // 10, C), lambda i: (i, 0)),
        out_shape=jax.ShapeDtypeStruct((N, C), jnp.float32),
    )(x, t1, q, Wc, b2)
    return out


# speedup vs baseline: 1.4642x; 1.0476x over previous
import functools

import jax
import jax.numpy as jnp
from jax import lax
from jax.experimental import pallas as pl
from jax.experimental.pallas import tpu as pltpu
from jax.experimental.pallas import tpu_sc as plsc

N = 10000
E = 320000
C = 128
NP = 10240
CHUNK = 80
DEG_ROWS_PER_TILE = (E // CHUNK) // 16
W_ROWS_PER_WORKER = (E // 16) // 32
SPMM_ROWS_PER_WORKER = (E // CHUNK) // 32
NPT = NP // 16

_MESH = plsc.VectorSubcoreMesh(
    core_axis_name="c", subcore_axis_name="s", num_cores=2, num_subcores=16
)


@functools.partial(
    pl.kernel,
    out_type=jax.ShapeDtypeStruct((32, SPMM_ROWS_PER_WORKER, CHUNK), jnp.float32),
    mesh=_MESH,
    compiler_params=pltpu.CompilerParams(needs_layout_passes=False, use_tc_tiling_on_sc=False),
    scratch_types=[
        pltpu.VMEM((SPMM_ROWS_PER_WORKER, CHUNK), jnp.int32),
        pltpu.VMEM((SPMM_ROWS_PER_WORKER, CHUNK), jnp.int32),
        pltpu.VMEM((SPMM_ROWS_PER_WORKER, CHUNK), jnp.float32),
        pltpu.VMEM((CHUNK,), jnp.float32),
        pltpu.VMEM((NPT,), jnp.float32),
        pltpu.VMEM((NP,), jnp.float32),
        pltpu.VMEM_SHARED((NP,), jnp.float32),
        pltpu.VMEM_SHARED((NP,), jnp.float32),
        pltpu.SemaphoreType.DMA,
    ],
)
def _prepass(row3_hbm, col3_hbm, a_hbm,
             rowv80, colv80, av, onesv, degv, disv, deg_sh, dis_sh, dsem):
    c = lax.axis_index("c")
    s = lax.axis_index("s")
    wid = c * 16 + s

    for i in range(CHUNK // 16):
        onesv[pl.ds(i * 16, 16)] = jnp.full((16,), 1.0, jnp.float32)
    for i in range(NPT // 16):
        degv[pl.ds(i * 16, 16)] = jnp.zeros((16,), jnp.float32)
    pltpu.sync_copy(degv, deg_sh.at[pl.ds(s * NPT, NPT)])
    plsc.subcore_barrier()

    DEPTH = 8

    def deg_issue(j, carry):
        pltpu.async_copy(onesv, deg_sh.at[rowv80.at[j + DEPTH]], dsem, add=True)
        pltpu.make_async_copy(onesv, deg_sh.at[rowv80.at[j]], dsem).wait()
        return carry

    def deg_drain(j, carry):
        pltpu.make_async_copy(onesv, deg_sh.at[rowv80.at[j]], dsem).wait()
        return carry

    for half in range(2):
        pltpu.sync_copy(row3_hbm.at[s * 2 + half], rowv80)
        for u in range(DEPTH):
            pltpu.async_copy(onesv, deg_sh.at[rowv80.at[u]], dsem, add=True)
        lax.fori_loop(0, SPMM_ROWS_PER_WORKER - DEPTH, deg_issue, 0)
        lax.fori_loop(SPMM_ROWS_PER_WORKER - DEPTH, SPMM_ROWS_PER_WORKER,
                      deg_drain, 0)
    plsc.subcore_barrier()

    pltpu.sync_copy(deg_sh.at[pl.ds(s * NPT, NPT)], degv)
    for i in range(NPT // 16):
        d = degv[pl.ds(i * 16, 16)]
        dsafe = jnp.maximum(d, 1.0)
        sq = (dsafe + 1.0) * 0.5
        for _ in range(14):
            sq = (sq + dsafe / sq) * 0.5
        degv[pl.ds(i * 16, 16)] = jnp.where(
            d > 0.0, 1.0 / sq, jnp.zeros((16,), jnp.float32))
    pltpu.sync_copy(degv, dis_sh.at[pl.ds(s * NPT, NPT)])
    plsc.subcore_barrier()

    pltpu.sync_copy(dis_sh, disv)
    pltpu.sync_copy(row3_hbm.at[wid], rowv80)
    pltpu.sync_copy(col3_hbm.at[wid], colv80)

    def w_body(j, carry):
        for k in range(CHUNK // 16):
            dr = plsc.load_gather(disv, [rowv80[j, pl.ds(k * 16, 16)]])
            dc = plsc.load_gather(disv, [colv80[j, pl.ds(k * 16, 16)]])
            av[j, pl.ds(k * 16, 16)] = -(dr * dc) - 1.0
        return carry

    lax.fori_loop(0, SPMM_ROWS_PER_WORKER, w_body, 0)
    pltpu.sync_copy(av, a_hbm.at[wid])


@functools.partial(
    pl.kernel,
    out_type=jax.ShapeDtypeStruct((2, 16, 625, C), jnp.float32),
    mesh=_MESH,
    compiler_params=pltpu.CompilerParams(needs_layout_passes=False, use_tc_tiling_on_sc=False),
    scratch_types=[
        pltpu.VMEM((SPMM_ROWS_PER_WORKER, 2, CHUNK // 2), jnp.int32),
        pltpu.VMEM((SPMM_ROWS_PER_WORKER, 2, CHUNK // 2), jnp.int32),
        pltpu.VMEM((SPMM_ROWS_PER_WORKER, CHUNK), jnp.float32),
        pltpu.VMEM((CHUNK, C), jnp.float32),
        pltpu.VMEM((CHUNK, C), jnp.float32),
        pltpu.VMEM_SHARED((N, C), jnp.float32),
        pltpu.SemaphoreType.DMA,
        pltpu.SemaphoreType.DMA,
        pltpu.SemaphoreType.DMA,
        pltpu.SemaphoreType.DMA,
        pltpu.SemaphoreType.DMA,
        pltpu.SemaphoreType.DMA,
    ],
)
def _spmm(h_hbm, col3_hbm, row3_hbm, a3_hbm, part_hbm,
          colv, rowv, av, gbuf0, gbuf1, acc_sh,
          sem0a, sem0b, sem1a, sem1b, ssem0, ssem1):
    c = lax.axis_index("c")
    s = lax.axis_index("s")
    wid = c * 16 + s

    for i in range(CHUNK):
        for r in range(C // 16):
            gbuf0[i, pl.ds(r * 16, 16)] = jnp.zeros((16,), jnp.float32)
    for i in range(7):
        pltpu.sync_copy(gbuf0, acc_sh.at[pl.ds(s * 625 + i * 80, 80)])
    pltpu.sync_copy(gbuf0.at[pl.ds(0, 65)], acc_sh.at[pl.ds(s * 625 + 560, 65)])
    plsc.subcore_barrier()

    pltpu.sync_copy(col3_hbm.at[wid], colv)
    pltpu.sync_copy(row3_hbm.at[wid], rowv)
    pltpu.sync_copy(a3_hbm.at[wid], av)

    HALF = CHUNK // 2

    def scale_half(m, h, gbuf):
        for off, lo in ((0, 0), (16, 0), (24, 8)):
            avv = av[m, pl.ds(h * HALF + off, 16)]
            for t in range(lo, 16):
                jj = h * HALF + off + t
                aa = avv[t]
                for r in range(C // 16):
                    gbuf[jj, pl.ds(r * 16, 16)] = gbuf[jj, pl.ds(r * 16, 16)] * aa

    def issue_gather(m, gbuf, sema, semb):
        pltpu.async_copy(h_hbm.at[colv.at[m, 0]], gbuf.at[pl.ds(0, HALF)], sema)
        pltpu.async_copy(h_hbm.at[colv.at[m, 1]], gbuf.at[pl.ds(HALF, HALF)], semb)

    def process(m, gbuf, sema, semb, ssem):
        pltpu.make_async_copy(h_hbm.at[colv.at[m, 0]], gbuf.at[pl.ds(0, HALF)],
                              sema).wait()
        scale_half(m, 0, gbuf)
        pltpu.async_copy(gbuf.at[pl.ds(0, HALF)], acc_sh.at[rowv.at[m, 0]],
                         ssem, add=True)
        pltpu.make_async_copy(h_hbm.at[colv.at[m, 1]], gbuf.at[pl.ds(HALF, HALF)],
                              semb).wait()
        scale_half(m, 1, gbuf)
        pltpu.async_copy(gbuf.at[pl.ds(HALF, HALF)], acc_sh.at[rowv.at[m, 1]],
                         ssem, add=True)
        pltpu.make_async_copy(gbuf.at[pl.ds(0, HALF)], acc_sh.at[rowv.at[m, 0]],
                              ssem).wait()
        pltpu.make_async_copy(gbuf.at[pl.ds(HALF, HALF)], acc_sh.at[rowv.at[m, 1]],
                              ssem).wait()

    issue_gather(0, gbuf0, sem0a, sem0b)

    def body(j, carry):
        m0 = 2 * j
        issue_gather(m0 + 1, gbuf1, sem1a, sem1b)
        process(m0, gbuf0, sem0a, sem0b, ssem0)
        issue_gather(m0 + 2, gbuf0, sem0a, sem0b)
        process(m0 + 1, gbuf1, sem1a, sem1b, ssem1)
        return carry

    lax.fori_loop(0, (SPMM_ROWS_PER_WORKER - 1) // 2, body, 0)
    process(SPMM_ROWS_PER_WORKER - 1, gbuf0, sem0a, sem0b, ssem0)
    plsc.subcore_barrier()

    pltpu.sync_copy(acc_sh.at[pl.ds(s * 625, 625)], part_hbm.at[c, s])


def _combine_body(p_ref, o_ref):
    o_ref[...] = p_ref[0] + p_ref[1]


def _final_body(x_ref, t1_ref, q_ref, w_ref, b_ref, o_ref):
    s2 = q_ref[0] + q_ref[1]
    acc = jnp.dot(x_ref[...], w_ref[0], preferred_element_type=jnp.float32)
    acc = acc + jnp.dot(t1_ref[...], w_ref[1], preferred_element_type=jnp.float32)
    acc = acc + jnp.dot(s2, w_ref[2], preferred_element_type=jnp.float32)
    o_ref[...] = acc + b_ref[...]


def kernel(x, edge_index, W, b):
    row = edge_index[0]
    col = edge_index[1]
    row3 = row.reshape(32, SPMM_ROWS_PER_WORKER, CHUNK)
    col3 = col.reshape(32, SPMM_ROWS_PER_WORKER, CHUNK)

    a3 = _prepass(row3, col3)

    row4 = row.reshape(32, SPMM_ROWS_PER_WORKER, 2, CHUNK // 2)
    col4 = col.reshape(32, SPMM_ROWS_PER_WORKER, 2, CHUNK // 2)
    p = _spmm(x, col4, row4, a3).reshape(2, N, C)
    t1 = pl.pallas_call(
        _combine_body,
        grid=(10,),
        in_specs=[pl.BlockSpec((2, N // 10, C), lambda i: (0, i, 0))],
        out_specs=pl.BlockSpec((N // 10, C), lambda i: (i, 0)),
        out_shape=jax.ShapeDtypeStruct((N, C), jnp.float32),
    )(p)

    q = _spmm(t1, col4, row4, a3).reshape(2, N, C)

    Wc = jnp.stack([W[0] - W[2], W[1], 2.0 * W[2]])
    b2 = b.reshape(1, C)
    out = pl.pallas_call(
        _final_body,
        grid=(10,),
        in_specs=[
            pl.BlockSpec((N // 10, C), lambda i: (i, 0)),
            pl.BlockSpec((N // 10, C), lambda i: (i, 0)),
            pl.BlockSpec((2, N // 10, C), lambda i: (0, i, 0)),
            pl.BlockSpec((3, C, C), lambda i: (0, 0, 0)),
            pl.BlockSpec((1, C), lambda i: (0, 0)),
        ],
        out_specs=pl.BlockSpec((N // 10, C), lambda i: (i, 0)),
        out_shape=jax.ShapeDtypeStruct((N, C), jnp.float32),
    )(x, t1, q, Wc, b2)
    return out


# speedup vs baseline: 1.4651x; 1.0006x over previous
import functools

import jax
import jax.numpy as jnp
from jax import lax
from jax.experimental import pallas as pl
from jax.experimental.pallas import tpu as pltpu
from jax.experimental.pallas import tpu_sc as plsc

N = 10000
E = 320000
C = 128
NP = 10240
CHUNK = 80
SPMM_ROWS_PER_WORKER = (E // CHUNK) // 32
NPT = NP // 16

_MESH = plsc.VectorSubcoreMesh(
    core_axis_name="c", subcore_axis_name="s", num_cores=2, num_subcores=16
)


@functools.partial(
    pl.kernel,
    out_type=jax.ShapeDtypeStruct((32, SPMM_ROWS_PER_WORKER, CHUNK), jnp.float32),
    mesh=_MESH,
    compiler_params=pltpu.CompilerParams(needs_layout_passes=False, use_tc_tiling_on_sc=False),
    scratch_types=[
        pltpu.VMEM((SPMM_ROWS_PER_WORKER, CHUNK), jnp.int32),
        pltpu.VMEM((SPMM_ROWS_PER_WORKER, CHUNK), jnp.int32),
        pltpu.VMEM((SPMM_ROWS_PER_WORKER, CHUNK), jnp.float32),
        pltpu.VMEM((CHUNK,), jnp.float32),
        pltpu.VMEM((NPT,), jnp.float32),
        pltpu.VMEM((NP,), jnp.float32),
        pltpu.VMEM_SHARED((NP,), jnp.float32),
        pltpu.VMEM_SHARED((NP,), jnp.float32),
        pltpu.SemaphoreType.DMA,
    ],
)
def _prepass(row3_hbm, col3_hbm, a_hbm,
             rowv80, colv80, av, onesv, degv, disv, deg_sh, dis_sh, dsem):
    c = lax.axis_index("c")
    s = lax.axis_index("s")
    wid = c * 16 + s

    for i in range(CHUNK // 16):
        onesv[pl.ds(i * 16, 16)] = jnp.full((16,), 1.0, jnp.float32)
    for i in range(NPT // 16):
        degv[pl.ds(i * 16, 16)] = jnp.zeros((16,), jnp.float32)
    pltpu.sync_copy(degv, deg_sh.at[pl.ds(s * NPT, NPT)])
    plsc.subcore_barrier()

    DEPTH = 8

    def deg_issue(j, carry):
        pltpu.async_copy(onesv, deg_sh.at[rowv80.at[j + DEPTH]], dsem, add=True)
        pltpu.make_async_copy(onesv, deg_sh.at[rowv80.at[j]], dsem).wait()
        return carry

    def deg_drain(j, carry):
        pltpu.make_async_copy(onesv, deg_sh.at[rowv80.at[j]], dsem).wait()
        return carry

    for half in range(2):
        pltpu.sync_copy(row3_hbm.at[s * 2 + half], rowv80)
        for u in range(DEPTH):
            pltpu.async_copy(onesv, deg_sh.at[rowv80.at[u]], dsem, add=True)
        lax.fori_loop(0, SPMM_ROWS_PER_WORKER - DEPTH, deg_issue, 0)
        lax.fori_loop(SPMM_ROWS_PER_WORKER - DEPTH, SPMM_ROWS_PER_WORKER,
                      deg_drain, 0)
    plsc.subcore_barrier()

    pltpu.sync_copy(deg_sh.at[pl.ds(s * NPT, NPT)], degv)
    for i in range(NPT // 16):
        d = degv[pl.ds(i * 16, 16)]
        dsafe = jnp.maximum(d, 1.0)
        sq = (dsafe + 1.0) * 0.5
        for _ in range(14):
            sq = (sq + dsafe / sq) * 0.5
        degv[pl.ds(i * 16, 16)] = jnp.where(
            d > 0.0, 1.0 / sq, jnp.zeros((16,), jnp.float32))
    pltpu.sync_copy(degv, dis_sh.at[pl.ds(s * NPT, NPT)])
    plsc.subcore_barrier()

    pltpu.sync_copy(dis_sh, disv)
    pltpu.sync_copy(row3_hbm.at[wid], rowv80)
    pltpu.sync_copy(col3_hbm.at[wid], colv80)

    def w_body(j, carry):
        for k in range(CHUNK // 16):
            dr = plsc.load_gather(disv, [rowv80[j, pl.ds(k * 16, 16)]])
            dc = plsc.load_gather(disv, [colv80[j, pl.ds(k * 16, 16)]])
            av[j, pl.ds(k * 16, 16)] = -(dr * dc) - 1.0
        return carry

    lax.fori_loop(0, SPMM_ROWS_PER_WORKER, w_body, 0)
    pltpu.sync_copy(av, a_hbm.at[wid])


@functools.partial(
    pl.kernel,
    out_type=jax.ShapeDtypeStruct((2, 16, 625, C), jnp.float32),
    mesh=_MESH,
    compiler_params=pltpu.CompilerParams(needs_layout_passes=False, use_tc_tiling_on_sc=False),
    scratch_types=[
        pltpu.VMEM((SPMM_ROWS_PER_WORKER, 2, CHUNK // 2), jnp.int32),
        pltpu.VMEM((SPMM_ROWS_PER_WORKER, 2, CHUNK // 2), jnp.int32),
        pltpu.VMEM((SPMM_ROWS_PER_WORKER, CHUNK), jnp.float32),
        pltpu.VMEM((CHUNK, C), jnp.float32),
        pltpu.VMEM((CHUNK, C), jnp.float32),
        pltpu.VMEM_SHARED((N, C), jnp.float32),
        pltpu.SemaphoreType.DMA,
        pltpu.SemaphoreType.DMA,
        pltpu.SemaphoreType.DMA,
        pltpu.SemaphoreType.DMA,
        pltpu.SemaphoreType.DMA,
        pltpu.SemaphoreType.DMA,
    ],
)
def _spmm(h_hbm, col3_hbm, row3_hbm, a3_hbm, part_hbm,
          colv, rowv, av, gbuf0, gbuf1, acc_sh,
          sem0a, sem0b, sem1a, sem1b, ssem0, ssem1):
    c = lax.axis_index("c")
    s = lax.axis_index("s")
    wid = c * 16 + s

    for i in range(CHUNK):
        for r in range(C // 16):
            gbuf0[i, pl.ds(r * 16, 16)] = jnp.zeros((16,), jnp.float32)
    for i in range(7):
        pltpu.sync_copy(gbuf0, acc_sh.at[pl.ds(s * 625 + i * 80, 80)])
    pltpu.sync_copy(gbuf0.at[pl.ds(0, 65)], acc_sh.at[pl.ds(s * 625 + 560, 65)])
    plsc.subcore_barrier()

    pltpu.sync_copy(col3_hbm.at[wid], colv)
    pltpu.sync_copy(row3_hbm.at[wid], rowv)
    pltpu.sync_copy(a3_hbm.at[wid], av)

    HALF = CHUNK // 2

    def scale_half(m, h, gbuf):
        for off, lo in ((0, 0), (16, 0), (24, 8)):
            avv = av[m, pl.ds(h * HALF + off, 16)]
            for t in range(lo, 16):
                jj = h * HALF + off + t
                aa = avv[t]
                for r in range(C // 16):
                    gbuf[jj, pl.ds(r * 16, 16)] = gbuf[jj, pl.ds(r * 16, 16)] * aa

    def issue_gather(m, gbuf, sema, semb):
        pltpu.async_copy(h_hbm.at[colv.at[m, 0]], gbuf.at[pl.ds(0, HALF)], sema)
        pltpu.async_copy(h_hbm.at[colv.at[m, 1]], gbuf.at[pl.ds(HALF, HALF)], semb)

    def process(m, gbuf, sema, semb, ssem):
        pltpu.make_async_copy(h_hbm.at[colv.at[m, 0]], gbuf.at[pl.ds(0, HALF)],
                              sema).wait()
        scale_half(m, 0, gbuf)
        pltpu.async_copy(gbuf.at[pl.ds(0, HALF)], acc_sh.at[rowv.at[m, 0]],
                         ssem, add=True)
        pltpu.make_async_copy(h_hbm.at[colv.at[m, 1]], gbuf.at[pl.ds(HALF, HALF)],
                              semb).wait()
        scale_half(m, 1, gbuf)
        pltpu.async_copy(gbuf.at[pl.ds(HALF, HALF)], acc_sh.at[rowv.at[m, 1]],
                         ssem, add=True)
        pltpu.make_async_copy(gbuf.at[pl.ds(0, HALF)], acc_sh.at[rowv.at[m, 0]],
                              ssem).wait()
        pltpu.make_async_copy(gbuf.at[pl.ds(HALF, HALF)], acc_sh.at[rowv.at[m, 1]],
                              ssem).wait()

    issue_gather(0, gbuf0, sem0a, sem0b)

    def body(j, carry):
        m0 = 2 * j
        issue_gather(m0 + 1, gbuf1, sem1a, sem1b)
        process(m0, gbuf0, sem0a, sem0b, ssem0)
        issue_gather(m0 + 2, gbuf0, sem0a, sem0b)
        process(m0 + 1, gbuf1, sem1a, sem1b, ssem1)
        return carry

    lax.fori_loop(0, (SPMM_ROWS_PER_WORKER - 1) // 2, body, 0)
    process(SPMM_ROWS_PER_WORKER - 1, gbuf0, sem0a, sem0b, ssem0)
    plsc.subcore_barrier()

    pltpu.sync_copy(acc_sh.at[pl.ds(s * 625, 625)], part_hbm.at[c, s])


def _combine_body(p_ref, o_ref):
    o_ref[...] = p_ref[0] + p_ref[1]


def _final_body(x_ref, t1_ref, q_ref, w_ref, b_ref, o_ref):
    s2 = q_ref[0] + q_ref[1]
    acc = jnp.dot(x_ref[...], w_ref[0], preferred_element_type=jnp.float32)
    acc = acc + jnp.dot(t1_ref[...], w_ref[1], preferred_element_type=jnp.float32)
    acc = acc + jnp.dot(s2, w_ref[2], preferred_element_type=jnp.float32)
    o_ref[...] = acc + b_ref[...]


def kernel(x, edge_index, W, b):
    row = edge_index[0]
    col = edge_index[1]
    row3 = row.reshape(32, SPMM_ROWS_PER_WORKER, CHUNK)
    col3 = col.reshape(32, SPMM_ROWS_PER_WORKER, CHUNK)

    a3 = _prepass(row3, col3)

    row4 = row.reshape(32, SPMM_ROWS_PER_WORKER, 2, CHUNK // 2)
    col4 = col.reshape(32, SPMM_ROWS_PER_WORKER, 2, CHUNK // 2)
    p = _spmm(x, col4, row4, a3).reshape(2, N, C)
    t1 = pl.pallas_call(
        _combine_body,
        grid=(10,),
        in_specs=[pl.BlockSpec((2, N // 10, C), lambda i: (0, i, 0))],
        out_specs=pl.BlockSpec((N // 10, C), lambda i: (i, 0)),
        out_shape=jax.ShapeDtypeStruct((N, C), jnp.float32),
    )(p)

    q = _spmm(t1, col4, row4, a3).reshape(2, N, C)

    Wc = jnp.stack([W[0] - W[2], W[1], 2.0 * W[2]])
    b2 = b.reshape(1, C)
    out = pl.pallas_call(
        _final_body,
        grid=(10,),
        in_specs=[
            pl.BlockSpec((N // 10, C), lambda i: (i, 0)),
            pl.BlockSpec((N // 10, C), lambda i: (i, 0)),
            pl.BlockSpec((2, N // 10, C), lambda i: (0, i, 0)),
            pl.BlockSpec((3, C, C), lambda i: (0, 0, 0)),
            pl.BlockSpec((1, C), lambda i: (0, 0)),
        ],
        out_specs=pl.BlockSpec((N // 10, C), lambda i: (i, 0)),
        out_shape=jax.ShapeDtypeStruct((N, C), jnp.float32),
    )(x, t1, q, Wc, b2)
    return out
